# Initial kernel scaffold; baseline (speedup 1.0000x reference)
import functools

import jax
import jax.numpy as jnp
from jax import lax
from jax.experimental import pallas as pl
from jax.experimental.pallas import tpu as pltpu
from jax.experimental.pallas import tpu_sc as plsc

N = 10000
D = 128
NPAD = 10240
NC, NS, L = 2, 16, 16
NW = NC * NS
C = 128
ROWS_PER_TILE = NPAD // NS


def _cdiv(a, b):
    return (a + b - 1) // b


_GRID = 8
_RB = NPAD // _GRID


def _proj_body(x_ref, w_ref, av_s_ref, av_d_ref, h_ref, as_ref, ad_ref):
    h = jnp.dot(x_ref[...], w_ref[...], preferred_element_type=jnp.float32)
    h_ref[...] = h
    as_ref[...] = jnp.sum(h * av_s_ref[...][None, :], axis=1)
    ad_ref[...] = jnp.sum(h * av_d_ref[...][None, :], axis=1)


def _project(x, w, av_s, av_d):
    return pl.pallas_call(
        _proj_body,
        grid=(_GRID,),
        in_specs=[
            pl.BlockSpec((_RB, D), lambda i: (i, 0)),
            pl.BlockSpec((D, D), lambda i: (0, 0)),
            pl.BlockSpec((D,), lambda i: (0,)),
            pl.BlockSpec((D,), lambda i: (0,)),
        ],
        out_specs=[
            pl.BlockSpec((_RB, D), lambda i: (i, 0)),
            pl.BlockSpec((_RB,), lambda i: (i,)),
            pl.BlockSpec((_RB,), lambda i: (i,)),
        ],
        out_shape=[
            jax.ShapeDtypeStruct((NPAD, D), jnp.float32),
            jax.ShapeDtypeStruct((NPAD,), jnp.float32),
            jax.ShapeDtypeStruct((NPAD,), jnp.float32),
        ],
    )(x, w, av_s, av_d)


def _combine_body(accp_ref, sp_ref, b_ref, w_ref, av_s_ref, av_d_ref,
                  h_ref, as_ref, ad_ref):
    s = jnp.sum(sp_ref[...], axis=0)
    o = (accp_ref[0] + accp_ref[1]) / (s[:, None] + 1e-16) + b_ref[...][None, :]
    e = jnp.where(o > 0, o, jnp.expm1(o))
    h = jnp.dot(e, w_ref[...], preferred_element_type=jnp.float32)
    h_ref[...] = h
    as_ref[...] = jnp.sum(h * av_s_ref[...][None, :], axis=1)
    ad_ref[...] = jnp.sum(h * av_d_ref[...][None, :], axis=1)


def _combine_project(accp, sp, b, w, av_s, av_d):
    return pl.pallas_call(
        _combine_body,
        grid=(_GRID,),
        in_specs=[
            pl.BlockSpec((NC, _RB, D), lambda i: (0, i, 0)),
            pl.BlockSpec((NW, _RB), lambda i: (0, i)),
            pl.BlockSpec((D,), lambda i: (0,)),
            pl.BlockSpec((D, D), lambda i: (0, 0)),
            pl.BlockSpec((D,), lambda i: (0,)),
            pl.BlockSpec((D,), lambda i: (0,)),
        ],
        out_specs=[
            pl.BlockSpec((_RB, D), lambda i: (i, 0)),
            pl.BlockSpec((_RB,), lambda i: (i,)),
            pl.BlockSpec((_RB,), lambda i: (i,)),
        ],
        out_shape=[
            jax.ShapeDtypeStruct((NPAD, D), jnp.float32),
            jax.ShapeDtypeStruct((NPAD,), jnp.float32),
            jax.ShapeDtypeStruct((NPAD,), jnp.float32),
        ],
    )(accp, sp, b, w, av_s, av_d)


def _final_body(accp_ref, sp_ref, b_ref, out_ref):
    s = jnp.sum(sp_ref[...], axis=0)
    out_ref[...] = ((accp_ref[0] + accp_ref[1]) / (s[:, None] + 1e-16)
                    + b_ref[...][None, :])


def _final(accp, sp, b):
    return pl.pallas_call(
        _final_body,
        grid=(_GRID,),
        in_specs=[
            pl.BlockSpec((NC, _RB, D), lambda i: (0, i, 0)),
            pl.BlockSpec((NW, _RB), lambda i: (0, i)),
            pl.BlockSpec((D,), lambda i: (0,)),
        ],
        out_specs=pl.BlockSpec((_RB, D), lambda i: (i, 0)),
        out_shape=jax.ShapeDtypeStruct((NPAD, D), jnp.float32),
    )(accp, sp, b)


def _make_edge_phase(e_real, gpt):
    ept = gpt * C
    mesh = plsc.VectorSubcoreMesh(core_axis_name="c", subcore_axis_name="s")

    @functools.partial(
        pl.kernel,
        out_type=[
            jax.ShapeDtypeStruct((NC, NPAD, D), jnp.float32),
            jax.ShapeDtypeStruct((NW, NPAD), jnp.float32),
        ],
        mesh=mesh,
        scratch_types=[
            pltpu.VMEM((NPAD,), jnp.float32),
            pltpu.VMEM((NPAD,), jnp.float32),
            pltpu.VMEM((NPAD,), jnp.float32),
            pltpu.VMEM((gpt, C), jnp.int32),
            pltpu.VMEM((gpt, C), jnp.int32),
            pltpu.VMEM((C,), jnp.float32),
            pltpu.VMEM((C, D), jnp.float32),
            pltpu.VMEM_SHARED((NPAD, D), jnp.float32),
            pltpu.SemaphoreType.DMA,
        ],
    )
    def edge_phase(h_hbm, asrc_hbm, adst_hbm, srct_hbm, dstt_hbm,
                   accp_hbm, sp_hbm,
                   as_v, ad_v, s_v, src_t, dst_t, w_buf, rows, acc, sem):
        cid = lax.axis_index("c")
        sid = lax.axis_index("s")
        wid = cid * NS + sid

        zeros16 = jnp.zeros((L,), jnp.float32)

        @pl.loop(0, C)
        def _(j):
            for c in range(D // L):
                rows[j, pl.ds(c * L, L)] = zeros16

        for b in range(ROWS_PER_TILE // C):
            pltpu.sync_copy(rows, acc.at[pl.ds(sid * ROWS_PER_TILE + b * C, C)])

        @pl.loop(0, NPAD // L)
        def _(j):
            s_v[pl.ds(j * L, L)] = zeros16

        pltpu.sync_copy(asrc_hbm, as_v)
        pltpu.sync_copy(adst_hbm, ad_v)
        pltpu.sync_copy(srct_hbm.at[wid], src_t)
        pltpu.sync_copy(dstt_hbm.at[wid], dst_t)

        plsc.subcore_barrier()

        lanes = lax.iota(jnp.int32, L)

        @pl.loop(0, gpt)
        def _(g):
            gather = pltpu.async_copy(h_hbm.at[src_t.at[g]], rows, sem)

            base = wid * ept + g * C
            for k in range(C // L):
                sv = src_t[g, pl.ds(k * L, L)]
                dv = dst_t[g, pl.ds(k * L, L)]
                av = plsc.load_gather(as_v, [sv])
                bv = plsc.load_gather(ad_v, [dv])
                e = av + bv
                e = jnp.where(e >= 0, e, e * jnp.float32(0.2))
                w = jnp.exp(e)
                valid = (base + k * L + lanes) < e_real
                w = jnp.where(valid, w, jnp.float32(0.0))
                w_buf[pl.ds(k * L, L)] = w
                plsc.addupdate_scatter(s_v, [dv], w)

            gather.wait()

            @pl.loop(0, C)
            def _(j):
                wj = plsc.load_gather(w_buf, [jnp.broadcast_to(j, (L,))])
                for c in range(D // L):
                    sl = pl.ds(c * L, L)
                    rows[j, sl] = rows[j, sl] * wj

            pltpu.sync_copy(rows, acc.at[dst_t.at[g]], add=True)

        plsc.subcore_barrier()

        pltpu.sync_copy(s_v, sp_hbm.at[wid])
        pltpu.sync_copy(acc.at[pl.ds(sid * ROWS_PER_TILE, ROWS_PER_TILE)],
                        accp_hbm.at[cid].at[pl.ds(sid * ROWS_PER_TILE,
                                                  ROWS_PER_TILE)])

    return edge_phase


def kernel(x, edge_index, W1, a_src1, a_dst1, b1, W2, a_src2, a_dst2, b2):
    n = x.shape[0]
    e = edge_index.shape[1]
    e_real = e + n
    gpt = _cdiv(e_real, NW * C)
    epad = gpt * C * NW

    x_pad = jnp.pad(x, ((0, NPAD - n), (0, 0)))
    loop_idx = jnp.arange(n, dtype=edge_index.dtype)
    pad_idx = jnp.zeros((epad - e_real,), edge_index.dtype)
    src_t = jnp.concatenate([edge_index[0], loop_idx, pad_idx]).reshape(NW, gpt, C)
    dst_t = jnp.concatenate([edge_index[1], loop_idx, pad_idx]).reshape(NW, gpt, C)

    edge_phase = _make_edge_phase(e_real, gpt)

    h1, as1, ad1 = _project(x_pad, W1, a_src1.reshape(-1), a_dst1.reshape(-1))
    accp1, sp1 = edge_phase(h1, as1, ad1, src_t, dst_t)
    h2, as2, ad2 = _combine_project(accp1, sp1, b1, W2,
                                    a_src2.reshape(-1), a_dst2.reshape(-1))
    accp2, sp2 = edge_phase(h2, as2, ad2, src_t, dst_t)
    out = _final(accp2, sp2, b2)
    return out[:n]


# speedup vs baseline: 18.1618x; 18.1618x over previous
import dataclasses
import functools

import jax
import jax.numpy as jnp
from jax import lax
from jax.experimental import pallas as pl
from jax.experimental.pallas import tpu as pltpu
from jax.experimental.pallas import tpu_sc as plsc

N = 10000
D = 128
NPAD = 10240
NC, NS, L = 2, 16, 16
NW = NC * NS
C = 128
ROWS_PER_TILE = NPAD // NS


def _cdiv(a, b):
    return (a + b - 1) // b


_GRID = 10
_RB = NPAD // _GRID


def _proj_body(x_ref, w_ref, av_s_ref, av_d_ref, h_ref, as_ref, ad_ref):
    h = jnp.dot(x_ref[...], w_ref[...], preferred_element_type=jnp.float32)
    h_ref[...] = h
    as_ref[...] = jnp.sum(h * av_s_ref[...][None, :], axis=1)
    ad_ref[...] = jnp.sum(h * av_d_ref[...][None, :], axis=1)


def _project(x, w, av_s, av_d):
    return pl.pallas_call(
        _proj_body,
        grid=(_GRID,),
        in_specs=[
            pl.BlockSpec((_RB, D), lambda i: (i, 0)),
            pl.BlockSpec((D, D), lambda i: (0, 0)),
            pl.BlockSpec((D,), lambda i: (0,)),
            pl.BlockSpec((D,), lambda i: (0,)),
        ],
        out_specs=[
            pl.BlockSpec((_RB, D), lambda i: (i, 0)),
            pl.BlockSpec((_RB,), lambda i: (i,)),
            pl.BlockSpec((_RB,), lambda i: (i,)),
        ],
        out_shape=[
            jax.ShapeDtypeStruct((NPAD, D), jnp.float32),
            jax.ShapeDtypeStruct((NPAD,), jnp.float32),
            jax.ShapeDtypeStruct((NPAD,), jnp.float32),
        ],
    )(x, w, av_s, av_d)


def _combine_body(accp_ref, sp_ref, b_ref, w_ref, av_s_ref, av_d_ref,
                  h_ref, as_ref, ad_ref):
    s = jnp.sum(sp_ref[...], axis=0)
    o = (accp_ref[0] + accp_ref[1]) / (s[:, None] + 1e-16) + b_ref[...][None, :]
    e = jnp.where(o > 0, o, jnp.exp(o) - 1.0)
    h = jnp.dot(e, w_ref[...], preferred_element_type=jnp.float32)
    h_ref[...] = h
    as_ref[...] = jnp.sum(h * av_s_ref[...][None, :], axis=1)
    ad_ref[...] = jnp.sum(h * av_d_ref[...][None, :], axis=1)


def _combine_project(accp, sp, b, w, av_s, av_d):
    return pl.pallas_call(
        _combine_body,
        grid=(_GRID,),
        in_specs=[
            pl.BlockSpec((NC, _RB, D), lambda i: (0, i, 0)),
            pl.BlockSpec((NW, _RB), lambda i: (0, i)),
            pl.BlockSpec((D,), lambda i: (0,)),
            pl.BlockSpec((D, D), lambda i: (0, 0)),
            pl.BlockSpec((D,), lambda i: (0,)),
            pl.BlockSpec((D,), lambda i: (0,)),
        ],
        out_specs=[
            pl.BlockSpec((_RB, D), lambda i: (i, 0)),
            pl.BlockSpec((_RB,), lambda i: (i,)),
            pl.BlockSpec((_RB,), lambda i: (i,)),
        ],
        out_shape=[
            jax.ShapeDtypeStruct((NPAD, D), jnp.float32),
            jax.ShapeDtypeStruct((NPAD,), jnp.float32),
            jax.ShapeDtypeStruct((NPAD,), jnp.float32),
        ],
    )(accp, sp, b, w, av_s, av_d)


def _final_body(accp_ref, sp_ref, b_ref, out_ref):
    s = jnp.sum(sp_ref[...], axis=0)
    out_ref[...] = ((accp_ref[0] + accp_ref[1]) / (s[:, None] + 1e-16)
                    + b_ref[...][None, :])


def _final(accp, sp, b):
    return pl.pallas_call(
        _final_body,
        grid=(_GRID,),
        in_specs=[
            pl.BlockSpec((NC, _RB, D), lambda i: (0, i, 0)),
            pl.BlockSpec((NW, _RB), lambda i: (0, i)),
            pl.BlockSpec((D,), lambda i: (0,)),
        ],
        out_specs=pl.BlockSpec((_RB, D), lambda i: (i, 0)),
        out_shape=jax.ShapeDtypeStruct((NPAD, D), jnp.float32),
    )(accp, sp, b)


def _make_edge_phase(e_real, gpt):
    ept = gpt * C
    mesh = plsc.VectorSubcoreMesh(core_axis_name="c", subcore_axis_name="s")
    cp = pltpu.CompilerParams()
    if "needs_layout_passes" in pltpu.CompilerParams.__dataclass_fields__:
        cp = dataclasses.replace(cp, needs_layout_passes=False)

    @functools.partial(
        pl.kernel,
        compiler_params=cp,
        out_type=[
            jax.ShapeDtypeStruct((NC, NPAD, D), jnp.float32),
            jax.ShapeDtypeStruct((NW, NPAD), jnp.float32),
        ],
        mesh=mesh,
        scratch_types=[
            pltpu.VMEM((NPAD,), jnp.float32),
            pltpu.VMEM((NPAD,), jnp.float32),
            pltpu.VMEM((NPAD,), jnp.float32),
            pltpu.VMEM((2, C), jnp.int32),
            pltpu.VMEM((2, C), jnp.int32),
            pltpu.VMEM((C,), jnp.float32),
            pltpu.VMEM((C, D), jnp.float32),
            pltpu.VMEM_SHARED((NPAD, D), jnp.float32),
            pltpu.SemaphoreType.DMA,
            pltpu.SemaphoreType.DMA,
            pltpu.SemaphoreType.DMA,
        ],
    )
    def edge_phase(h_hbm, asrc_hbm, adst_hbm, idx_hbm,
                   accp_hbm, sp_hbm,
                   as_v, ad_v, s_v, idx_a, idx_b, w_buf, rows, acc,
                   sem_a, sem_b, sem_g):
        cid = lax.axis_index("c")
        sid = lax.axis_index("s")
        wid = cid * NS + sid

        zeros16 = jnp.zeros((L,), jnp.float32)

        @pl.loop(0, C)
        def _(j):
            for c in range(D // L):
                rows[j, pl.ds(c * L, L)] = zeros16

        for b in range(ROWS_PER_TILE // C):
            pltpu.sync_copy(rows, acc.at[pl.ds(sid * ROWS_PER_TILE + b * C, C)])

        @pl.loop(0, NPAD // L)
        def _(j):
            s_v[pl.ds(j * L, L)] = zeros16

        pltpu.sync_copy(asrc_hbm, as_v)
        pltpu.sync_copy(adst_hbm, ad_v)
        pltpu.async_copy(idx_hbm.at[wid].at[0], idx_a, sem_a)
        pltpu.async_copy(idx_hbm.at[wid].at[1], idx_b, sem_b)

        plsc.subcore_barrier()

        lanes = lax.iota(jnp.int32, L)

        def process_chunk(g, ibuf, isem):
            pltpu.make_async_copy(idx_hbm.at[wid].at[g], ibuf, isem).wait()
            gather = pltpu.async_copy(h_hbm.at[ibuf.at[0]], rows, sem_g)

            base = wid * ept + g * C
            for k in range(C // L):
                sv = ibuf[0, pl.ds(k * L, L)]
                dv = ibuf[1, pl.ds(k * L, L)]
                av = plsc.load_gather(as_v, [sv])
                bv = plsc.load_gather(ad_v, [dv])
                e = av + bv
                e = jnp.where(e >= 0, e, e * jnp.float32(0.2))
                w = jnp.exp(e)
                valid = (base + k * L + lanes) < e_real
                w = jnp.where(valid, w, jnp.float32(0.0))
                w_buf[pl.ds(k * L, L)] = w
                plsc.addupdate_scatter(s_v, [dv], w)

            gather.wait()

            @pl.loop(0, C)
            def _(j):
                wj = plsc.load_gather(w_buf, [jnp.broadcast_to(j, (L,))])
                for c in range(D // L):
                    sl = pl.ds(c * L, L)
                    rows[j, sl] = rows[j, sl] * wj

            pltpu.sync_copy(rows, acc.at[ibuf.at[1]], add=True)

            @pl.when(g + 2 < gpt)
            def _():
                pltpu.async_copy(idx_hbm.at[wid].at[g + 2], ibuf, isem)

        @pl.loop(0, gpt, step=2)
        def _(g):
            process_chunk(g, idx_a, sem_a)
            process_chunk(g + 1, idx_b, sem_b)

        plsc.subcore_barrier()

        pltpu.sync_copy(s_v, sp_hbm.at[wid])
        pltpu.sync_copy(acc.at[pl.ds(sid * ROWS_PER_TILE, ROWS_PER_TILE)],
                        accp_hbm.at[cid].at[pl.ds(sid * ROWS_PER_TILE,
                                                  ROWS_PER_TILE)])

    return edge_phase


def kernel(x, edge_index, W1, a_src1, a_dst1, b1, W2, a_src2, a_dst2, b2):
    n = x.shape[0]
    e = edge_index.shape[1]
    e_real = e + n
    gpt = _cdiv(e_real, NW * C)
    gpt += gpt % 2
    epad = gpt * C * NW

    x_pad = jnp.pad(x, ((0, NPAD - n), (0, 0)))
    loop_idx = jnp.arange(n, dtype=edge_index.dtype)
    pad_idx = jnp.zeros((epad - e_real,), edge_index.dtype)
    src_t = jnp.concatenate([edge_index[0], loop_idx, pad_idx]).reshape(NW, gpt, C)
    dst_t = jnp.concatenate([edge_index[1], loop_idx, pad_idx]).reshape(NW, gpt, C)
    idx_t = jnp.stack([src_t, dst_t], axis=2)

    edge_phase = _make_edge_phase(e_real, gpt)

    h1, as1, ad1 = _project(x_pad, W1, a_src1.reshape(-1), a_dst1.reshape(-1))
    accp1, sp1 = edge_phase(h1, as1, ad1, idx_t)
    h2, as2, ad2 = _combine_project(accp1, sp1, b1, W2,
                                    a_src2.reshape(-1), a_dst2.reshape(-1))
    accp2, sp2 = edge_phase(h2, as2, ad2, idx_t)
    out = _final(accp2, sp2, b2)
    return out[:n]


# speedup vs baseline: 18.4523x; 1.0160x over previous
import dataclasses
import functools

import jax
import jax.numpy as jnp
from jax import lax
from jax.experimental import pallas as pl
from jax.experimental.pallas import tpu as pltpu
from jax.experimental.pallas import tpu_sc as plsc

N = 10000
D = 128
NPAD = 10240
NC, NS, L = 2, 16, 16
NW = NC * NS
C = 128
ROWS_PER_TILE = NPAD // NS


def _cdiv(a, b):
    return (a + b - 1) // b


_GRID = 10
_RB = NPAD // _GRID


def _proj_body(x_ref, w_ref, av_s_ref, av_d_ref, h_ref, as_ref, ad_ref):
    h = jnp.dot(x_ref[...], w_ref[...], preferred_element_type=jnp.float32)
    h_ref[...] = h
    as_ref[...] = jnp.sum(h * av_s_ref[...][None, :], axis=1)
    ad_ref[...] = jnp.sum(h * av_d_ref[...][None, :], axis=1)


def _project(x, w, av_s, av_d):
    return pl.pallas_call(
        _proj_body,
        grid=(_GRID,),
        in_specs=[
            pl.BlockSpec((_RB, D), lambda i: (i, 0)),
            pl.BlockSpec((D, D), lambda i: (0, 0)),
            pl.BlockSpec((D,), lambda i: (0,)),
            pl.BlockSpec((D,), lambda i: (0,)),
        ],
        out_specs=[
            pl.BlockSpec((_RB, D), lambda i: (i, 0)),
            pl.BlockSpec((_RB,), lambda i: (i,)),
            pl.BlockSpec((_RB,), lambda i: (i,)),
        ],
        out_shape=[
            jax.ShapeDtypeStruct((NPAD, D), jnp.float32),
            jax.ShapeDtypeStruct((NPAD,), jnp.float32),
            jax.ShapeDtypeStruct((NPAD,), jnp.float32),
        ],
    )(x, w, av_s, av_d)


def _combine_body(accp_ref, sp_ref, b_ref, w_ref, av_s_ref, av_d_ref,
                  h_ref, as_ref, ad_ref):
    s = jnp.sum(sp_ref[...], axis=0)
    o = (accp_ref[0] + accp_ref[1]) / (s[:, None] + 1e-16) + b_ref[...][None, :]
    e = jnp.where(o > 0, o, jnp.exp(o) - 1.0)
    h = jnp.dot(e, w_ref[...], preferred_element_type=jnp.float32)
    h_ref[...] = h
    as_ref[...] = jnp.sum(h * av_s_ref[...][None, :], axis=1)
    ad_ref[...] = jnp.sum(h * av_d_ref[...][None, :], axis=1)


def _combine_project(accp, sp, b, w, av_s, av_d):
    return pl.pallas_call(
        _combine_body,
        grid=(_GRID,),
        in_specs=[
            pl.BlockSpec((NC, _RB, D), lambda i: (0, i, 0)),
            pl.BlockSpec((NW, _RB), lambda i: (0, i)),
            pl.BlockSpec((D,), lambda i: (0,)),
            pl.BlockSpec((D, D), lambda i: (0, 0)),
            pl.BlockSpec((D,), lambda i: (0,)),
            pl.BlockSpec((D,), lambda i: (0,)),
        ],
        out_specs=[
            pl.BlockSpec((_RB, D), lambda i: (i, 0)),
            pl.BlockSpec((_RB,), lambda i: (i,)),
            pl.BlockSpec((_RB,), lambda i: (i,)),
        ],
        out_shape=[
            jax.ShapeDtypeStruct((NPAD, D), jnp.float32),
            jax.ShapeDtypeStruct((NPAD,), jnp.float32),
            jax.ShapeDtypeStruct((NPAD,), jnp.float32),
        ],
    )(accp, sp, b, w, av_s, av_d)


def _final_body(accp_ref, sp_ref, b_ref, out_ref):
    s = jnp.sum(sp_ref[...], axis=0)
    out_ref[...] = ((accp_ref[0] + accp_ref[1]) / (s[:, None] + 1e-16)
                    + b_ref[...][None, :])


def _final(accp, sp, b):
    return pl.pallas_call(
        _final_body,
        grid=(_GRID,),
        in_specs=[
            pl.BlockSpec((NC, _RB, D), lambda i: (0, i, 0)),
            pl.BlockSpec((NW, _RB), lambda i: (0, i)),
            pl.BlockSpec((D,), lambda i: (0,)),
        ],
        out_specs=pl.BlockSpec((_RB, D), lambda i: (i, 0)),
        out_shape=jax.ShapeDtypeStruct((NPAD, D), jnp.float32),
    )(accp, sp, b)


def _make_edge_phase(e_real, gpt):
    ept = gpt * C
    mesh = plsc.VectorSubcoreMesh(core_axis_name="c", subcore_axis_name="s")
    cp = pltpu.CompilerParams()
    if "needs_layout_passes" in pltpu.CompilerParams.__dataclass_fields__:
        cp = dataclasses.replace(cp, needs_layout_passes=False)

    @functools.partial(
        pl.kernel,
        compiler_params=cp,
        out_type=[
            jax.ShapeDtypeStruct((NC, NPAD, D), jnp.float32),
            jax.ShapeDtypeStruct((NW, NPAD), jnp.float32),
        ],
        mesh=mesh,
        scratch_types=[
            pltpu.VMEM((NPAD,), jnp.float32),
            pltpu.VMEM((NPAD,), jnp.float32),
            pltpu.VMEM((NPAD,), jnp.float32),
            pltpu.VMEM((2, C), jnp.int32),
            pltpu.VMEM((2, C), jnp.int32),
            pltpu.VMEM((C,), jnp.float32),
            pltpu.VMEM((C, D), jnp.float32),
            pltpu.VMEM_SHARED((NPAD, D), jnp.float32),
            pltpu.SemaphoreType.DMA,
            pltpu.SemaphoreType.DMA,
            pltpu.SemaphoreType.DMA,
        ],
    )
    def edge_phase(h_hbm, asrc_hbm, adst_hbm, idx_hbm,
                   accp_hbm, sp_hbm,
                   as_v, ad_v, s_v, idx_a, idx_b, w_buf, rows, acc,
                   sem_a, sem_b, sem_g):
        cid = lax.axis_index("c")
        sid = lax.axis_index("s")
        wid = cid * NS + sid

        zeros16 = jnp.zeros((L,), jnp.float32)

        @pl.loop(0, C)
        def _(j):
            for c in range(D // L):
                rows[j, pl.ds(c * L, L)] = zeros16

        for b in range(ROWS_PER_TILE // C):
            pltpu.sync_copy(rows, acc.at[pl.ds(sid * ROWS_PER_TILE + b * C, C)])

        @pl.loop(0, NPAD // L)
        def _(j):
            s_v[pl.ds(j * L, L)] = zeros16

        pltpu.sync_copy(asrc_hbm, as_v)
        pltpu.sync_copy(adst_hbm, ad_v)
        pltpu.async_copy(idx_hbm.at[wid].at[0], idx_a, sem_a)
        pltpu.async_copy(idx_hbm.at[wid].at[1], idx_b, sem_b)

        plsc.subcore_barrier()

        lanes = lax.iota(jnp.int32, L)

        def process_chunk(g, ibuf, isem):
            pltpu.make_async_copy(idx_hbm.at[wid].at[g], ibuf, isem).wait()
            gather = pltpu.async_copy(h_hbm.at[ibuf.at[0]], rows, sem_g)

            base = wid * ept + g * C
            for k in range(C // L):
                sv = ibuf[0, pl.ds(k * L, L)]
                dv = ibuf[1, pl.ds(k * L, L)]
                av = plsc.load_gather(as_v, [sv])
                bv = plsc.load_gather(ad_v, [dv])
                e = av + bv
                e = jnp.where(e >= 0, e, e * jnp.float32(0.2))
                w = jnp.exp(e)
                valid = (base + k * L + lanes) < e_real
                w = jnp.where(valid, w, jnp.float32(0.0))
                w_buf[pl.ds(k * L, L)] = w
                plsc.addupdate_scatter(s_v, [dv], w)

            gather.wait()

            @pl.loop(0, C, unroll=4)
            def _(j):
                wj = plsc.load_gather(w_buf, [jnp.broadcast_to(j, (L,))])
                for c in range(D // L):
                    sl = pl.ds(c * L, L)
                    rows[j, sl] = rows[j, sl] * wj

            pltpu.sync_copy(rows, acc.at[ibuf.at[1]], add=True)

            @pl.when(g + 2 < gpt)
            def _():
                pltpu.async_copy(idx_hbm.at[wid].at[g + 2], ibuf, isem)

        @pl.loop(0, gpt, step=2)
        def _(g):
            process_chunk(g, idx_a, sem_a)
            process_chunk(g + 1, idx_b, sem_b)

        plsc.subcore_barrier()

        pltpu.sync_copy(s_v, sp_hbm.at[wid])
        pltpu.sync_copy(acc.at[pl.ds(sid * ROWS_PER_TILE, ROWS_PER_TILE)],
                        accp_hbm.at[cid].at[pl.ds(sid * ROWS_PER_TILE,
                                                  ROWS_PER_TILE)])

    return edge_phase


def kernel(x, edge_index, W1, a_src1, a_dst1, b1, W2, a_src2, a_dst2, b2):
    n = x.shape[0]
    e = edge_index.shape[1]
    e_real = e + n
    gpt = _cdiv(e_real, NW * C)
    gpt += gpt % 2
    epad = gpt * C * NW

    x_pad = jnp.pad(x, ((0, NPAD - n), (0, 0)))
    loop_idx = jnp.arange(n, dtype=edge_index.dtype)
    pad_idx = jnp.zeros((epad - e_real,), edge_index.dtype)
    src_t = jnp.concatenate([edge_index[0], loop_idx, pad_idx]).reshape(NW, gpt, C)
    dst_t = jnp.concatenate([edge_index[1], loop_idx, pad_idx]).reshape(NW, gpt, C)
    idx_t = jnp.stack([src_t, dst_t], axis=2)

    edge_phase = _make_edge_phase(e_real, gpt)

    h1, as1, ad1 = _project(x_pad, W1, a_src1.reshape(-1), a_dst1.reshape(-1))
    accp1, sp1 = edge_phase(h1, as1, ad1, idx_t)
    h2, as2, ad2 = _combine_project(accp1, sp1, b1, W2,
                                    a_src2.reshape(-1), a_dst2.reshape(-1))
    accp2, sp2 = edge_phase(h2, as2, ad2, idx_t)
    out = _final(accp2, sp2, b2)
    return out[:n]


# speedup vs baseline: 29.5186x; 1.5997x over previous
import dataclasses
import functools

import jax
import jax.numpy as jnp
from jax import lax
from jax.experimental import pallas as pl
from jax.experimental.pallas import tpu as pltpu
from jax.experimental.pallas import tpu_sc as plsc

N = 10000
D = 128
NPAD = 10240
NC, NS, L = 2, 16, 16
NW = NC * NS
C = 128
H = C // 2
ROWS_PER_TILE = NPAD // NS


def _cdiv(a, b):
    return (a + b - 1) // b


_GRID = 10
_RB = NPAD // _GRID


def _proj_body(x_ref, w_ref, av_s_ref, av_d_ref, h_ref, as_ref, ad_ref):
    h = jnp.dot(x_ref[...], w_ref[...], preferred_element_type=jnp.float32)
    h_ref[...] = h
    as_ref[...] = jnp.sum(h * av_s_ref[...][None, :], axis=1)
    ad_ref[...] = jnp.sum(h * av_d_ref[...][None, :], axis=1)


def _project(x, w, av_s, av_d):
    return pl.pallas_call(
        _proj_body,
        grid=(_GRID,),
        in_specs=[
            pl.BlockSpec((_RB, D), lambda i: (i, 0)),
            pl.BlockSpec((D, D), lambda i: (0, 0)),
            pl.BlockSpec((D,), lambda i: (0,)),
            pl.BlockSpec((D,), lambda i: (0,)),
        ],
        out_specs=[
            pl.BlockSpec((_RB, D), lambda i: (i, 0)),
            pl.BlockSpec((_RB,), lambda i: (i,)),
            pl.BlockSpec((_RB,), lambda i: (i,)),
        ],
        out_shape=[
            jax.ShapeDtypeStruct((NPAD, D), jnp.float32),
            jax.ShapeDtypeStruct((NPAD,), jnp.float32),
            jax.ShapeDtypeStruct((NPAD,), jnp.float32),
        ],
    )(x, w, av_s, av_d)


def _combine_body(accp_ref, sp_ref, b_ref, w_ref, av_s_ref, av_d_ref,
                  h_ref, as_ref, ad_ref):
    s = jnp.sum(sp_ref[...], axis=0)
    o = (accp_ref[0] + accp_ref[1]) / (s[:, None] + 1e-16) + b_ref[...][None, :]
    e = jnp.where(o > 0, o, jnp.exp(o) - 1.0)
    h = jnp.dot(e, w_ref[...], preferred_element_type=jnp.float32)
    h_ref[...] = h
    as_ref[...] = jnp.sum(h * av_s_ref[...][None, :], axis=1)
    ad_ref[...] = jnp.sum(h * av_d_ref[...][None, :], axis=1)


def _combine_project(accp, sp, b, w, av_s, av_d):
    return pl.pallas_call(
        _combine_body,
        grid=(_GRID,),
        in_specs=[
            pl.BlockSpec((NC, _RB, D), lambda i: (0, i, 0)),
            pl.BlockSpec((NW, _RB), lambda i: (0, i)),
            pl.BlockSpec((D,), lambda i: (0,)),
            pl.BlockSpec((D, D), lambda i: (0, 0)),
            pl.BlockSpec((D,), lambda i: (0,)),
            pl.BlockSpec((D,), lambda i: (0,)),
        ],
        out_specs=[
            pl.BlockSpec((_RB, D), lambda i: (i, 0)),
            pl.BlockSpec((_RB,), lambda i: (i,)),
            pl.BlockSpec((_RB,), lambda i: (i,)),
        ],
        out_shape=[
            jax.ShapeDtypeStruct((NPAD, D), jnp.float32),
            jax.ShapeDtypeStruct((NPAD,), jnp.float32),
            jax.ShapeDtypeStruct((NPAD,), jnp.float32),
        ],
    )(accp, sp, b, w, av_s, av_d)


def _final_body(accp_ref, sp_ref, b_ref, out_ref):
    s = jnp.sum(sp_ref[...], axis=0)
    out_ref[...] = ((accp_ref[0] + accp_ref[1]) / (s[:, None] + 1e-16)
                    + b_ref[...][None, :])


def _final(accp, sp, b):
    return pl.pallas_call(
        _final_body,
        grid=(_GRID,),
        in_specs=[
            pl.BlockSpec((NC, _RB, D), lambda i: (0, i, 0)),
            pl.BlockSpec((NW, _RB), lambda i: (0, i)),
            pl.BlockSpec((D,), lambda i: (0,)),
        ],
        out_specs=pl.BlockSpec((_RB, D), lambda i: (i, 0)),
        out_shape=jax.ShapeDtypeStruct((NPAD, D), jnp.float32),
    )(accp, sp, b)


def _make_edge_phase(e_real, gpt):
    ept = gpt * C
    mesh = plsc.VectorSubcoreMesh(core_axis_name="c", subcore_axis_name="s")
    cp = pltpu.CompilerParams()
    if "needs_layout_passes" in pltpu.CompilerParams.__dataclass_fields__:
        cp = dataclasses.replace(cp, needs_layout_passes=False)

    @functools.partial(
        pl.kernel,
        compiler_params=cp,
        out_type=[
            jax.ShapeDtypeStruct((NC, NPAD, D), jnp.float32),
            jax.ShapeDtypeStruct((NW, NPAD), jnp.float32),
        ],
        mesh=mesh,
        scratch_types=[
            pltpu.VMEM((NPAD,), jnp.float32),
            pltpu.VMEM((NPAD,), jnp.float32),
            pltpu.VMEM((NPAD,), jnp.float32),
            pltpu.VMEM((4, H), jnp.int32),
            pltpu.VMEM((4, H), jnp.int32),
            pltpu.VMEM((4, H), jnp.int32),
            pltpu.VMEM((C,), jnp.float32),
            pltpu.VMEM((H, D), jnp.float32),
            pltpu.VMEM((H, D), jnp.float32),
            pltpu.VMEM_SHARED((NPAD, D), jnp.float32),
            pltpu.SemaphoreType.DMA,
            pltpu.SemaphoreType.DMA,
            pltpu.SemaphoreType.DMA,
            pltpu.SemaphoreType.DMA,
            pltpu.SemaphoreType.DMA,
            pltpu.SemaphoreType.DMA,
            pltpu.SemaphoreType.DMA,
        ],
    )
    def edge_phase(h_hbm, asrc_hbm, adst_hbm, idx_hbm,
                   accp_hbm, sp_hbm,
                   as_v, ad_v, s_v, idx_0, idx_1, idx_2, w_buf,
                   rows_a, rows_b, acc,
                   isem_0, isem_1, isem_2, gsem_a, gsem_b, ssem_a, ssem_b):
        cid = lax.axis_index("c")
        sid = lax.axis_index("s")
        wid = cid * NS + sid
        ibufs = (idx_0, idx_1, idx_2)
        isems = (isem_0, isem_1, isem_2)

        zeros16 = jnp.zeros((L,), jnp.float32)

        @pl.loop(0, H)
        def _(j):
            for c in range(D // L):
                rows_a[j, pl.ds(c * L, L)] = zeros16
                rows_b[j, pl.ds(c * L, L)] = zeros16

        for b in range(ROWS_PER_TILE // C):
            base_r = sid * ROWS_PER_TILE + b * C
            pltpu.sync_copy(rows_a, acc.at[pl.ds(base_r, H)])
            pltpu.sync_copy(rows_b, acc.at[pl.ds(base_r + H, H)])

        @pl.loop(0, NPAD // L)
        def _(j):
            s_v[pl.ds(j * L, L)] = zeros16

        pltpu.sync_copy(asrc_hbm, as_v)
        pltpu.sync_copy(adst_hbm, ad_v)
        pltpu.async_copy(idx_hbm.at[wid].at[0], idx_0, isem_0)
        pltpu.async_copy(idx_hbm.at[wid].at[1], idx_1, isem_1)

        plsc.subcore_barrier()

        lanes = lax.iota(jnp.int32, L)

        def scalar_phase(g, ibuf):
            base = wid * ept + g * C
            for k in range(C // L):
                sv = ibuf[k * L // H, pl.ds((k * L) % H, L)]
                dv = ibuf[2 + k * L // H, pl.ds((k * L) % H, L)]
                av = plsc.load_gather(as_v, [sv])
                bv = plsc.load_gather(ad_v, [dv])
                e = av + bv
                e = jnp.where(e >= 0, e, e * jnp.float32(0.2))
                w = jnp.exp(e)
                valid = (base + k * L + lanes) < e_real
                w = jnp.where(valid, w, jnp.float32(0.0))
                w_buf[pl.ds(k * L, L)] = w
                plsc.addupdate_scatter(s_v, [dv], w)

        def multiply(rbuf, half):
            @pl.loop(0, H, unroll=4)
            def _(j):
                wj = plsc.load_gather(w_buf, [jnp.broadcast_to(j + half * H,
                                                               (L,))])
                for c in range(D // L):
                    sl = pl.ds(c * L, L)
                    rbuf[j, sl] = rbuf[j, sl] * wj

        def process_chunk(g, ibuf, isem, pbuf, psem):
            pltpu.make_async_copy(idx_hbm.at[wid].at[g], ibuf, isem).wait()

            @pl.when(g > 0)
            def _():
                pltpu.make_async_copy(rows_a, acc.at[ibuf.at[2]],
                                      ssem_a).wait()
                pltpu.make_async_copy(rows_b, acc.at[ibuf.at[3]],
                                      ssem_b).wait()

            ga = pltpu.async_copy(h_hbm.at[ibuf.at[0]], rows_a, gsem_a)
            gb = pltpu.async_copy(h_hbm.at[ibuf.at[1]], rows_b, gsem_b)

            scalar_phase(g, ibuf)

            @pl.when(g + 2 < gpt)
            def _():
                pltpu.async_copy(idx_hbm.at[wid].at[g + 2], pbuf, psem)

            ga.wait()
            multiply(rows_a, 0)
            pltpu.async_copy(rows_a, acc.at[ibuf.at[2]], ssem_a, add=True)
            gb.wait()
            multiply(rows_b, 1)
            pltpu.async_copy(rows_b, acc.at[ibuf.at[3]], ssem_b, add=True)

        @pl.loop(0, gpt, step=3)
        def _(g):
            process_chunk(g, idx_0, isem_0, idx_2, isem_2)
            process_chunk(g + 1, idx_1, isem_1, idx_0, isem_0)
            process_chunk(g + 2, idx_2, isem_2, idx_1, isem_1)

        pltpu.make_async_copy(rows_a, acc.at[idx_2.at[2]], ssem_a).wait()
        pltpu.make_async_copy(rows_b, acc.at[idx_2.at[3]], ssem_b).wait()

        plsc.subcore_barrier()

        pltpu.sync_copy(s_v, sp_hbm.at[wid])
        pltpu.sync_copy(acc.at[pl.ds(sid * ROWS_PER_TILE, ROWS_PER_TILE)],
                        accp_hbm.at[cid].at[pl.ds(sid * ROWS_PER_TILE,
                                                  ROWS_PER_TILE)])

    return edge_phase


def kernel(x, edge_index, W1, a_src1, a_dst1, b1, W2, a_src2, a_dst2, b2):
    n = x.shape[0]
    e = edge_index.shape[1]
    e_real = e + n
    gpt = _cdiv(e_real, NW * C)
    gpt = _cdiv(gpt, 3) * 3
    epad = gpt * C * NW

    x_pad = jnp.pad(x, ((0, NPAD - n), (0, 0)))
    loop_idx = jnp.arange(n, dtype=edge_index.dtype)
    pad_idx = jnp.zeros((epad - e_real,), edge_index.dtype)
    src_t = jnp.concatenate([edge_index[0], loop_idx, pad_idx]).reshape(
        NW, gpt, 2, H)
    dst_t = jnp.concatenate([edge_index[1], loop_idx, pad_idx]).reshape(
        NW, gpt, 2, H)
    idx_t = jnp.concatenate([src_t, dst_t], axis=2)

    edge_phase = _make_edge_phase(e_real, gpt)

    h1, as1, ad1 = _project(x_pad, W1, a_src1.reshape(-1), a_dst1.reshape(-1))
    accp1, sp1 = edge_phase(h1, as1, ad1, idx_t)
    h2, as2, ad2 = _combine_project(accp1, sp1, b1, W2,
                                    a_src2.reshape(-1), a_dst2.reshape(-1))
    accp2, sp2 = edge_phase(h2, as2, ad2, idx_t)
    out = _final(accp2, sp2, b2)
    return out[:n]


# speedup vs baseline: 36.6012x; 1.2399x over previous
import dataclasses
import functools

import jax
import jax.numpy as jnp
from jax import lax
from jax.experimental import pallas as pl
from jax.experimental.pallas import tpu as pltpu
from jax.experimental.pallas import tpu_sc as plsc

N = 10000
D = 128
NPAD = 10240
NC, NS, L = 2, 16, 16
NW = NC * NS
C = 96
NQ = 3
Q = C // NQ
NV = 10112
ROWS_PER_TILE = NPAD // NS


def _cdiv(a, b):
    return (a + b - 1) // b


_GRID = 10
_RB = NPAD // _GRID


def _proj_body(x_ref, w_ref, av_s_ref, av_d_ref, h_ref, as_ref, ad_ref):
    h = jnp.dot(x_ref[...], w_ref[...], preferred_element_type=jnp.float32)
    h_ref[...] = h
    as_ref[...] = jnp.sum(h * av_s_ref[...][None, :], axis=1)
    ad_ref[...] = jnp.sum(h * av_d_ref[...][None, :], axis=1)


def _project(x, w, av_s, av_d):
    return pl.pallas_call(
        _proj_body,
        grid=(_GRID,),
        in_specs=[
            pl.BlockSpec((_RB, D), lambda i: (i, 0)),
            pl.BlockSpec((D, D), lambda i: (0, 0)),
            pl.BlockSpec((D,), lambda i: (0,)),
            pl.BlockSpec((D,), lambda i: (0,)),
        ],
        out_specs=[
            pl.BlockSpec((_RB, D), lambda i: (i, 0)),
            pl.BlockSpec((_RB,), lambda i: (i,)),
            pl.BlockSpec((_RB,), lambda i: (i,)),
        ],
        out_shape=[
            jax.ShapeDtypeStruct((NPAD, D), jnp.float32),
            jax.ShapeDtypeStruct((NPAD,), jnp.float32),
            jax.ShapeDtypeStruct((NPAD,), jnp.float32),
        ],
    )(x, w, av_s, av_d)


def _combine_body(accp_ref, sp_ref, b_ref, w_ref, av_s_ref, av_d_ref,
                  h_ref, as_ref, ad_ref):
    s = jnp.sum(sp_ref[...], axis=0)
    o = (accp_ref[0] + accp_ref[1]) / (s[:, None] + 1e-16) + b_ref[...][None, :]
    e = jnp.where(o > 0, o, jnp.exp(o) - 1.0)
    h = jnp.dot(e, w_ref[...], preferred_element_type=jnp.float32)
    h_ref[...] = h
    as_ref[...] = jnp.sum(h * av_s_ref[...][None, :], axis=1)
    ad_ref[...] = jnp.sum(h * av_d_ref[...][None, :], axis=1)


def _combine_project(accp, sp, b, w, av_s, av_d):
    return pl.pallas_call(
        _combine_body,
        grid=(_GRID,),
        in_specs=[
            pl.BlockSpec((NC, _RB, D), lambda i: (0, i, 0)),
            pl.BlockSpec((NW, _RB), lambda i: (0, i)),
            pl.BlockSpec((D,), lambda i: (0,)),
            pl.BlockSpec((D, D), lambda i: (0, 0)),
            pl.BlockSpec((D,), lambda i: (0,)),
            pl.BlockSpec((D,), lambda i: (0,)),
        ],
        out_specs=[
            pl.BlockSpec((_RB, D), lambda i: (i, 0)),
            pl.BlockSpec((_RB,), lambda i: (i,)),
            pl.BlockSpec((_RB,), lambda i: (i,)),
        ],
        out_shape=[
            jax.ShapeDtypeStruct((NPAD, D), jnp.float32),
            jax.ShapeDtypeStruct((NPAD,), jnp.float32),
            jax.ShapeDtypeStruct((NPAD,), jnp.float32),
        ],
    )(accp, sp, b, w, av_s, av_d)


def _final_body(accp_ref, sp_ref, b_ref, out_ref):
    s = jnp.sum(sp_ref[...], axis=0)
    out_ref[...] = ((accp_ref[0] + accp_ref[1]) / (s[:, None] + 1e-16)
                    + b_ref[...][None, :])


def _final(accp, sp, b):
    return pl.pallas_call(
        _final_body,
        grid=(_GRID,),
        in_specs=[
            pl.BlockSpec((NC, _RB, D), lambda i: (0, i, 0)),
            pl.BlockSpec((NW, _RB), lambda i: (0, i)),
            pl.BlockSpec((D,), lambda i: (0,)),
        ],
        out_specs=pl.BlockSpec((_RB, D), lambda i: (i, 0)),
        out_shape=jax.ShapeDtypeStruct((NPAD, D), jnp.float32),
    )(accp, sp, b)


def _make_edge_phase(e_real, gpt):
    ept = gpt * C
    mesh = plsc.VectorSubcoreMesh(core_axis_name="c", subcore_axis_name="s")
    cp = pltpu.CompilerParams()
    if "needs_layout_passes" in pltpu.CompilerParams.__dataclass_fields__:
        cp = dataclasses.replace(cp, needs_layout_passes=False)

    @functools.partial(
        pl.kernel,
        compiler_params=cp,
        out_type=[
            jax.ShapeDtypeStruct((NC, NPAD, D), jnp.float32),
            jax.ShapeDtypeStruct((NW, NPAD), jnp.float32),
        ],
        mesh=mesh,
        scratch_types=[
            pltpu.VMEM((NV,), jnp.float32),
            pltpu.VMEM((NV,), jnp.float32),
            pltpu.VMEM((NV,), jnp.float32),
            pltpu.VMEM((2 * NQ, Q), jnp.int32),
            pltpu.VMEM((2 * NQ, Q), jnp.int32),
            pltpu.VMEM((2 * NQ, Q), jnp.int32),
            pltpu.VMEM((C,), jnp.float32),
            pltpu.VMEM((Q, D), jnp.float32),
            pltpu.VMEM((Q, D), jnp.float32),
            pltpu.VMEM((Q, D), jnp.float32),
            pltpu.VMEM_SHARED((NPAD, D), jnp.float32),
            pltpu.SemaphoreType.DMA,
            pltpu.SemaphoreType.DMA,
            pltpu.SemaphoreType.DMA,
            pltpu.SemaphoreType.DMA,
            pltpu.SemaphoreType.DMA,
            pltpu.SemaphoreType.DMA,
            pltpu.SemaphoreType.DMA,
            pltpu.SemaphoreType.DMA,
            pltpu.SemaphoreType.DMA,
        ],
    )
    def edge_phase(h_hbm, asrc_hbm, adst_hbm, idx_hbm,
                   accp_hbm, sp_hbm,
                   as_v, ad_v, s_v, idx_0, idx_1, idx_2, w_buf,
                   rq_0, rq_1, rq_2, acc,
                   isem_0, isem_1, isem_2,
                   gsem_0, gsem_1, gsem_2,
                   ssem_0, ssem_1, ssem_2):
        cid = lax.axis_index("c")
        sid = lax.axis_index("s")
        wid = cid * NS + sid
        rqs = (rq_0, rq_1, rq_2)
        gsems = (gsem_0, gsem_1, gsem_2)
        ssems = (ssem_0, ssem_1, ssem_2)

        zeros16 = jnp.zeros((L,), jnp.float32)

        @pl.loop(0, Q)
        def _(j):
            for c in range(D // L):
                rq_0[j, pl.ds(c * L, L)] = zeros16

        @pl.loop(0, ROWS_PER_TILE // Q)
        def _(b):
            pltpu.sync_copy(rq_0,
                            acc.at[pl.ds(sid * ROWS_PER_TILE + b * Q, Q)])

        @pl.loop(0, NV // L)
        def _(j):
            s_v[pl.ds(j * L, L)] = zeros16

        pltpu.sync_copy(asrc_hbm.at[pl.ds(0, NV)], as_v)
        pltpu.sync_copy(adst_hbm.at[pl.ds(0, NV)], ad_v)
        pltpu.async_copy(idx_hbm.at[wid].at[0], idx_0, isem_0)
        pltpu.async_copy(idx_hbm.at[wid].at[1], idx_1, isem_1)

        plsc.subcore_barrier()

        lanes = lax.iota(jnp.int32, L)

        def scalar_phase(g, ibuf):
            base = wid * ept + g * C
            for k in range(C // L):
                row, off = (k * L) // Q, (k * L) % Q
                sv = ibuf[row, pl.ds(off, L)]
                dv = ibuf[NQ + row, pl.ds(off, L)]
                av = plsc.load_gather(as_v, [sv])
                bv = plsc.load_gather(ad_v, [dv])
                e = av + bv
                e = jnp.where(e >= 0, e, e * jnp.float32(0.2))
                w = jnp.exp(e)
                valid = (base + k * L + lanes) < e_real
                w = jnp.where(valid, w, jnp.float32(0.0))
                w_buf[pl.ds(k * L, L)] = w
                plsc.addupdate_scatter(s_v, [dv], w)

        def multiply(i):
            @pl.loop(0, Q, unroll=4)
            def _(j):
                wj = plsc.load_gather(w_buf, [jnp.broadcast_to(j + i * Q,
                                                               (L,))])
                for c in range(D // L):
                    sl = pl.ds(c * L, L)
                    rqs[i][j, sl] = rqs[i][j, sl] * wj

        def wait_scatter(i, ibuf):
            pltpu.make_async_copy(rqs[i], acc.at[ibuf.at[NQ + i]],
                                  ssems[i]).wait()

        def issue_gather(i, ibuf):
            pltpu.async_copy(h_hbm.at[ibuf.at[i]], rqs[i], gsems[i])

        def finish_quarter(i, ibuf):
            pltpu.make_async_copy(h_hbm.at[ibuf.at[i]], rqs[i],
                                  gsems[i]).wait()
            multiply(i)
            pltpu.async_copy(rqs[i], acc.at[ibuf.at[NQ + i]], ssems[i],
                             add=True)

        def process_chunk(g, ibuf, nbuf, nsem, pbuf, psem):
            scalar_phase(g, ibuf)

            @pl.when(g > 0)
            def _():
                wait_scatter(2, ibuf)
            issue_gather(2, ibuf)

            @pl.when(g + 2 < gpt)
            def _():
                pltpu.async_copy(idx_hbm.at[wid].at[g + 2], pbuf, psem)

            finish_quarter(0, ibuf)
            finish_quarter(1, ibuf)

            @pl.when(g + 1 < gpt)
            def _():
                pltpu.make_async_copy(idx_hbm.at[wid].at[g + 1], nbuf,
                                      nsem).wait()
                wait_scatter(0, nbuf)
                issue_gather(0, nbuf)
                wait_scatter(1, nbuf)
                issue_gather(1, nbuf)

            finish_quarter(2, ibuf)

        pltpu.make_async_copy(idx_hbm.at[wid].at[0], idx_0, isem_0).wait()
        issue_gather(0, idx_0)
        issue_gather(1, idx_0)

        @pl.loop(0, gpt, step=3)
        def _(g):
            process_chunk(g, idx_0, idx_1, isem_1, idx_2, isem_2)
            process_chunk(g + 1, idx_1, idx_2, isem_2, idx_0, isem_0)
            process_chunk(g + 2, idx_2, idx_0, isem_0, idx_1, isem_1)

        for i in range(NQ):
            wait_scatter(i, idx_2)

        plsc.subcore_barrier()

        pltpu.sync_copy(s_v, sp_hbm.at[wid].at[pl.ds(0, NV)])
        pltpu.sync_copy(acc.at[pl.ds(sid * ROWS_PER_TILE, ROWS_PER_TILE)],
                        accp_hbm.at[cid].at[pl.ds(sid * ROWS_PER_TILE,
                                                  ROWS_PER_TILE)])

    return edge_phase


def kernel(x, edge_index, W1, a_src1, a_dst1, b1, W2, a_src2, a_dst2, b2):
    n = x.shape[0]
    e = edge_index.shape[1]
    e_real = e + n
    gpt = _cdiv(e_real, NW * C)
    gpt = _cdiv(gpt, 3) * 3
    epad = gpt * C * NW

    x_pad = jnp.pad(x, ((0, NPAD - n), (0, 0)))
    loop_idx = jnp.arange(n, dtype=edge_index.dtype)
    pad_idx = jnp.zeros((epad - e_real,), edge_index.dtype)
    src_t = jnp.concatenate([edge_index[0], loop_idx, pad_idx]).reshape(
        NW, gpt, NQ, Q)
    dst_t = jnp.concatenate([edge_index[1], loop_idx, pad_idx]).reshape(
        NW, gpt, NQ, Q)
    idx_t = jnp.concatenate([src_t, dst_t], axis=2)

    edge_phase = _make_edge_phase(e_real, gpt)

    h1, as1, ad1 = _project(x_pad, W1, a_src1.reshape(-1), a_dst1.reshape(-1))
    accp1, sp1 = edge_phase(h1, as1, ad1, idx_t)
    h2, as2, ad2 = _combine_project(accp1, sp1, b1, W2,
                                    a_src2.reshape(-1), a_dst2.reshape(-1))
    accp2, sp2 = edge_phase(h2, as2, ad2, idx_t)
    out = _final(accp2, sp2, b2)
    return out[:n]
